# baseline (speedup 1.0000x reference)
import functools
import math

import jax
import jax.numpy as jnp
from jax import lax
from jax.experimental import pallas as pl
from jax.experimental.pallas import tpu as pltpu
from jax.experimental.pallas import tpu_sc as plsc


def _ssp(v):
    return jnp.maximum(v, 0.0) + jnp.log1p(jnp.exp(-jnp.abs(v))) - math.log(2.0)


def _in2f_body(x_ref, w_ref, o_ref):
    o_ref[...] = jnp.dot(x_ref[...], w_ref[...],
                         preferred_element_type=jnp.float32)


def _in2f(x, w):
    n, d = x.shape
    f = w.shape[1]
    blk = 2000
    return pl.pallas_call(
        _in2f_body,
        grid=(n // blk,),
        in_specs=[
            pl.BlockSpec((blk, d), lambda i: (i, 0)),
            pl.BlockSpec((d, f), lambda i: (0, 0)),
        ],
        out_specs=pl.BlockSpec((blk, f), lambda i: (i, 0)),
        out_shape=jax.ShapeDtypeStruct((n, f), jnp.float32),
    )(x, w)


_NC, _NS = 2, 16
_NW = _NC * _NS
_CH = 80


def _sc_gather(table, idx3):
    nw, nchunk, ch = idx3.shape
    n, f = table.shape
    e = nw * nchunk * ch
    per_w = nchunk * ch
    mesh = plsc.VectorSubcoreMesh(core_axis_name="c", subcore_axis_name="s")

    @functools.partial(
        pl.kernel,
        out_type=jax.ShapeDtypeStruct((e, f), jnp.float32),
        mesh=mesh,
        scratch_types=[
            pltpu.VMEM((nchunk, ch), jnp.int32),
            pltpu.VMEM((ch, f), jnp.float32),
            pltpu.SemaphoreType.DMA,
        ],
    )
    def k(table_hbm, idx_hbm, out_hbm, idx_v, rows_v, sem):
        wid = lax.axis_index("s") * _NC + lax.axis_index("c")
        base = wid * per_w
        pltpu.sync_copy(idx_hbm.at[wid], idx_v)

        def body(c, carry):
            pltpu.async_copy(table_hbm.at[idx_v.at[c]], rows_v, sem).wait()
            pltpu.sync_copy(rows_v, out_hbm.at[pl.ds(base + c * ch, ch)])
            return carry

        lax.fori_loop(0, nchunk, body, 0, unroll=False)

    return k(table, idx3)


def _fused_body(dre_ref, drm_ref, pm_ref, gat_ref,
                fw1_ref, fb1_ref, fw2_ref, fb2_ref,
                f2o_ref, f2ob_ref, ow_ref, ob_ref, o_ref,
                *, blk_n, nbh):
    h = jnp.dot(dre_ref[...], fw1_ref[...],
                preferred_element_type=jnp.float32) + fb1_ref[...]
    h = _ssp(h)
    w = jnp.dot(h, fw2_ref[...],
                preferred_element_type=jnp.float32) + fb2_ref[...]
    dr = drm_ref[...]
    cut = 0.5 * (jnp.cos(dr * (math.pi / 5.0)) + 1.0)
    cut = cut * (dr < 5.0).astype(jnp.float32) * pm_ref[...]
    z = gat_ref[...] * w * cut
    za = z.reshape(blk_n, nbh, z.shape[-1]).sum(axis=1)
    ya = _ssp(jnp.dot(za, f2o_ref[...],
                      preferred_element_type=jnp.float32) + f2ob_ref[...])
    o_ref[...] = jnp.dot(ya, ow_ref[...],
                         preferred_element_type=jnp.float32) + ob_ref[...]


def _fused(dre, drf, pmf, gathered, fw1, fb1, fw2, fb2, f2o, f2ob, ow, ob,
           n, nbh, d, f, g):
    blk_n = 400
    blk_e = blk_n * nbh
    grid = (n // blk_n,)
    body = functools.partial(_fused_body, blk_n=blk_n, nbh=nbh)
    return pl.pallas_call(
        body,
        grid=grid,
        in_specs=[
            pl.BlockSpec((blk_e, g), lambda i: (i, 0)),
            pl.BlockSpec((blk_e, 1), lambda i: (i, 0)),
            pl.BlockSpec((blk_e, 1), lambda i: (i, 0)),
            pl.BlockSpec((blk_e, f), lambda i: (i, 0)),
            pl.BlockSpec((g, f), lambda i: (0, 0)),
            pl.BlockSpec((1, f), lambda i: (0, 0)),
            pl.BlockSpec((f, f), lambda i: (0, 0)),
            pl.BlockSpec((1, f), lambda i: (0, 0)),
            pl.BlockSpec((f, d), lambda i: (0, 0)),
            pl.BlockSpec((1, d), lambda i: (0, 0)),
            pl.BlockSpec((d, d), lambda i: (0, 0)),
            pl.BlockSpec((1, d), lambda i: (0, 0)),
        ],
        out_specs=pl.BlockSpec((blk_n, d), lambda i: (i, 0)),
        out_shape=jax.ShapeDtypeStruct((n, d), jnp.float32),
    )(dre, drf, pmf, gathered, fw1, fb1, fw2, fb2, f2o, f2ob, ow, ob)


def kernel(x, dR, neighbors, pairwise_mask, dR_expanded,
           fW1, fb1, fW2, fb2, in2f_W, f2out_W, f2out_b, out_W, out_b):
    n, nbh = neighbors.shape
    d = x.shape[1]
    f = in2f_W.shape[1]
    g = dR_expanded.shape[2]
    e = n * nbh

    y = _in2f(x, in2f_W)

    per_w = e // _NW
    idx3 = neighbors.astype(jnp.int32).reshape(_NW, per_w // _CH, _CH)
    gathered = _sc_gather(y, idx3)

    dre = dR_expanded.reshape(e, g)
    drf = dR.reshape(e, 1)
    pmf = pairwise_mask.reshape(e, 1)
    return _fused(dre, drf, pmf, gathered,
                  fW1, fb1.reshape(1, f), fW2, fb2.reshape(1, f),
                  f2out_W, f2out_b.reshape(1, d), out_W, out_b.reshape(1, d),
                  n, nbh, d, f, g)


# speedup vs baseline: 2.5283x; 2.5283x over previous
import functools
import math

import jax
import jax.numpy as jnp
from jax import lax
from jax.experimental import pallas as pl
from jax.experimental.pallas import tpu as pltpu
from jax.experimental.pallas import tpu_sc as plsc


def _ssp(v):
    return jnp.maximum(v, 0.0) + jnp.log1p(jnp.exp(-jnp.abs(v))) - math.log(2.0)


def _in2f_body(x_ref, w_ref, o_ref):
    o_ref[...] = jnp.dot(x_ref[...], w_ref[...],
                         preferred_element_type=jnp.float32)


def _in2f(x, w):
    n, d = x.shape
    f = w.shape[1]
    blk = 2000
    return pl.pallas_call(
        _in2f_body,
        grid=(n // blk,),
        in_specs=[
            pl.BlockSpec((blk, d), lambda i: (i, 0)),
            pl.BlockSpec((d, f), lambda i: (0, 0)),
        ],
        out_specs=pl.BlockSpec((blk, f), lambda i: (i, 0)),
        out_shape=jax.ShapeDtypeStruct((n, f), jnp.float32),
    )(x, w)


_NC, _NS = 2, 16
_NW = _NC * _NS
_CH = 80


def _sc_gather(table, idx):
    (e,) = idx.shape
    n, f = table.shape
    per_w = e // _NW
    nchunk = per_w // _CH
    mesh = plsc.VectorSubcoreMesh(core_axis_name="c", subcore_axis_name="s")

    @functools.partial(
        pl.kernel,
        out_type=jax.ShapeDtypeStruct((e, f), jnp.float32),
        mesh=mesh,
        scratch_types=[
            pltpu.VMEM((per_w,), jnp.int32),
            pltpu.VMEM((_CH, f), jnp.float32),
            pltpu.SemaphoreType.DMA,
        ],
    )
    def k(table_hbm, idx_hbm, out_hbm, idx_v, rows_v, sem):
        wid = lax.axis_index("s") * _NC + lax.axis_index("c")
        base = wid * per_w
        pltpu.sync_copy(idx_hbm.at[pl.ds(base, per_w)], idx_v)

        def body(c, carry):
            pltpu.async_copy(
                table_hbm.at[idx_v.at[pl.ds(c * _CH, _CH)]], rows_v, sem
            ).wait()
            pltpu.sync_copy(rows_v, out_hbm.at[pl.ds(base + c * _CH, _CH)])
            return carry

        lax.fori_loop(0, nchunk, body, 0, unroll=False)

    return k(table, idx)


def _fused_body(dre_ref, dr_ref, pm_ref, gat_ref,
                fw1_ref, fb1_ref, fw2_ref, fb2_ref,
                f2o_ref, f2ob_ref, ow_ref, ob_ref, o_ref,
                *, blk_n, nbh):
    g = dre_ref.shape[-1]
    f = fw1_ref.shape[-1]
    dre = dre_ref[...].reshape(blk_n * nbh, g)
    h = jnp.dot(dre, fw1_ref[...],
                preferred_element_type=jnp.float32) + fb1_ref[...]
    h = _ssp(h)
    w = jnp.dot(h, fw2_ref[...],
                preferred_element_type=jnp.float32) + fb2_ref[...]
    dr = dr_ref[...]
    cut = 0.5 * (jnp.cos(dr * (math.pi / 5.0)) + 1.0)
    cut = cut * (dr < 5.0).astype(jnp.float32) * pm_ref[...]
    z = (gat_ref[...] * w).reshape(blk_n, nbh, f) * cut[:, :, None]
    za = z.sum(axis=1)
    ya = _ssp(jnp.dot(za, f2o_ref[...],
                      preferred_element_type=jnp.float32) + f2ob_ref[...])
    o_ref[...] = jnp.dot(ya, ow_ref[...],
                         preferred_element_type=jnp.float32) + ob_ref[...]


def _fused(dre, dr, pm, gathered, fw1, fb1, fw2, fb2, f2o, f2ob, ow, ob,
           n, nbh, d, f, g):
    blk_n = 400
    blk_e = blk_n * nbh
    grid = (n // blk_n,)
    body = functools.partial(_fused_body, blk_n=blk_n, nbh=nbh)
    return pl.pallas_call(
        body,
        grid=grid,
        in_specs=[
            pl.BlockSpec((blk_n, nbh, g), lambda i: (i, 0, 0)),
            pl.BlockSpec((blk_n, nbh), lambda i: (i, 0)),
            pl.BlockSpec((blk_n, nbh), lambda i: (i, 0)),
            pl.BlockSpec((blk_e, f), lambda i: (i, 0)),
            pl.BlockSpec((g, f), lambda i: (0, 0)),
            pl.BlockSpec((1, f), lambda i: (0, 0)),
            pl.BlockSpec((f, f), lambda i: (0, 0)),
            pl.BlockSpec((1, f), lambda i: (0, 0)),
            pl.BlockSpec((f, d), lambda i: (0, 0)),
            pl.BlockSpec((1, d), lambda i: (0, 0)),
            pl.BlockSpec((d, d), lambda i: (0, 0)),
            pl.BlockSpec((1, d), lambda i: (0, 0)),
        ],
        out_specs=pl.BlockSpec((blk_n, d), lambda i: (i, 0)),
        out_shape=jax.ShapeDtypeStruct((n, d), jnp.float32),
    )(dre, dr, pm, gathered, fw1, fb1, fw2, fb2, f2o, f2ob, ow, ob)


def kernel(x, dR, neighbors, pairwise_mask, dR_expanded,
           fW1, fb1, fW2, fb2, in2f_W, f2out_W, f2out_b, out_W, out_b):
    n, nbh = neighbors.shape
    d = x.shape[1]
    f = in2f_W.shape[1]
    g = dR_expanded.shape[2]

    y = _in2f(x, in2f_W)

    idx = neighbors.astype(jnp.int32).reshape(-1)
    gathered = _sc_gather(y, idx)

    return _fused(dR_expanded, dR, pairwise_mask, gathered,
                  fW1, fb1.reshape(1, f), fW2, fb2.reshape(1, f),
                  f2out_W, f2out_b.reshape(1, d), out_W, out_b.reshape(1, d),
                  n, nbh, d, f, g)


# speedup vs baseline: 2.8800x; 1.1391x over previous
import functools
import math

import jax
import jax.numpy as jnp
from jax import lax
from jax.experimental import pallas as pl
from jax.experimental.pallas import tpu as pltpu
from jax.experimental.pallas import tpu_sc as plsc


def _ssp(v):
    return jnp.maximum(v, 0.0) + jnp.log1p(jnp.exp(-jnp.abs(v))) - math.log(2.0)


def _in2f_body(x_ref, w_ref, o_ref):
    o_ref[...] = jnp.dot(x_ref[...], w_ref[...],
                         preferred_element_type=jnp.float32)


def _in2f(x, w):
    n, d = x.shape
    f = w.shape[1]
    blk = 2000
    return pl.pallas_call(
        _in2f_body,
        grid=(n // blk,),
        in_specs=[
            pl.BlockSpec((blk, d), lambda i: (i, 0)),
            pl.BlockSpec((d, f), lambda i: (0, 0)),
        ],
        out_specs=pl.BlockSpec((blk, f), lambda i: (i, 0)),
        out_shape=jax.ShapeDtypeStruct((n, f), jnp.float32),
    )(x, w)


_NC, _NS = 2, 16
_NW = _NC * _NS
_CH = 128
_K = 3
_PH = _CH * _K


def _sc_gather(table, idx):
    (e,) = idx.shape
    n, f = table.shape
    dt = table.dtype
    per_w = e // _NW
    nphase = per_w // _PH
    tail = per_w - nphase * _PH
    mesh = plsc.VectorSubcoreMesh(core_axis_name="c", subcore_axis_name="s")

    @functools.partial(
        pl.kernel,
        out_type=jax.ShapeDtypeStruct((e, f), dt),
        mesh=mesh,
        scratch_types=[
            pltpu.VMEM((per_w,), jnp.int32),
            pltpu.VMEM((_PH, f), dt),
            pltpu.VMEM((_PH, f), dt),
            pltpu.SemaphoreType.DMA,
            pltpu.SemaphoreType.DMA,
        ],
    )
    def k(table_hbm, idx_hbm, out_hbm, idx_v, buf_a, buf_b, sem_a, sem_b):
        wid = lax.axis_index("s") * _NC + lax.axis_index("c")
        base = wid * per_w
        pltpu.sync_copy(idx_hbm.at[pl.ds(base, per_w)], idx_v)

        def fire(buf, sem, p):
            for j in range(_K):
                pltpu.async_copy(
                    table_hbm.at[idx_v.at[pl.ds(p * _PH + j * _CH, _CH)]],
                    buf.at[pl.ds(j * _CH, _CH)], sem)

        def drain(buf, sem):
            pltpu.make_async_copy(
                table_hbm.at[pl.ds(0, _PH)], buf, sem).wait()

        def flush(buf, p):
            pltpu.sync_copy(buf, out_hbm.at[pl.ds(base + p * _PH, _PH)])

        last = nphase - 1
        fire(buf_a, sem_a, 0)

        def body(i, carry):
            p = 2 * i
            fire(buf_b, sem_b, p + 1)
            drain(buf_a, sem_a)
            flush(buf_a, p)
            fire(buf_a, sem_a, jnp.minimum(p + 2, last))
            drain(buf_b, sem_b)
            flush(buf_b, p + 1)
            return carry

        lax.fori_loop(0, nphase // 2, body, 0, unroll=False)
        drain(buf_a, sem_a)
        if tail:
            pltpu.async_copy(
                table_hbm.at[idx_v.at[pl.ds(nphase * _PH, tail)]],
                buf_a.at[pl.ds(0, tail)], sem_a).wait()
            pltpu.sync_copy(buf_a.at[pl.ds(0, tail)],
                            out_hbm.at[pl.ds(base + nphase * _PH, tail)])

    return k(table, idx)


def _fused_body(dre_ref, dr_ref, pm_ref, gat_ref,
                fw1_ref, fb1_ref, fw2_ref, fb2_ref,
                f2o_ref, f2ob_ref, ow_ref, ob_ref, o_ref,
                *, blk_n, nbh):
    g = dre_ref.shape[-1]
    f = fw1_ref.shape[-1]
    dre = dre_ref[...].reshape(blk_n * nbh, g)
    h = jnp.dot(dre, fw1_ref[...],
                preferred_element_type=jnp.float32) + fb1_ref[...]
    h = _ssp(h)
    w = jnp.dot(h, fw2_ref[...],
                preferred_element_type=jnp.float32) + fb2_ref[...]
    dr = dr_ref[...]
    cut = 0.5 * (jnp.cos(dr * (math.pi / 5.0)) + 1.0)
    cut = cut * (dr < 5.0).astype(jnp.float32) * pm_ref[...]
    z = (gat_ref[...] * w).reshape(blk_n, nbh, f) * cut[:, :, None]
    za = z.sum(axis=1)
    ya = _ssp(jnp.dot(za, f2o_ref[...],
                      preferred_element_type=jnp.float32) + f2ob_ref[...])
    o_ref[...] = jnp.dot(ya, ow_ref[...],
                         preferred_element_type=jnp.float32) + ob_ref[...]


def _fused(dre, dr, pm, gathered, fw1, fb1, fw2, fb2, f2o, f2ob, ow, ob,
           n, nbh, d, f, g):
    blk_n = 400
    blk_e = blk_n * nbh
    grid = (n // blk_n,)
    body = functools.partial(_fused_body, blk_n=blk_n, nbh=nbh)
    return pl.pallas_call(
        body,
        grid=grid,
        in_specs=[
            pl.BlockSpec((blk_n, nbh, g), lambda i: (i, 0, 0)),
            pl.BlockSpec((blk_n, nbh), lambda i: (i, 0)),
            pl.BlockSpec((blk_n, nbh), lambda i: (i, 0)),
            pl.BlockSpec((blk_e, f), lambda i: (i, 0)),
            pl.BlockSpec((g, f), lambda i: (0, 0)),
            pl.BlockSpec((1, f), lambda i: (0, 0)),
            pl.BlockSpec((f, f), lambda i: (0, 0)),
            pl.BlockSpec((1, f), lambda i: (0, 0)),
            pl.BlockSpec((f, d), lambda i: (0, 0)),
            pl.BlockSpec((1, d), lambda i: (0, 0)),
            pl.BlockSpec((d, d), lambda i: (0, 0)),
            pl.BlockSpec((1, d), lambda i: (0, 0)),
        ],
        out_specs=pl.BlockSpec((blk_n, d), lambda i: (i, 0)),
        out_shape=jax.ShapeDtypeStruct((n, d), jnp.float32),
    )(dre, dr, pm, gathered, fw1, fb1, fw2, fb2, f2o, f2ob, ow, ob)


def kernel(x, dR, neighbors, pairwise_mask, dR_expanded,
           fW1, fb1, fW2, fb2, in2f_W, f2out_W, f2out_b, out_W, out_b):
    n, nbh = neighbors.shape
    d = x.shape[1]
    f = in2f_W.shape[1]
    g = dR_expanded.shape[2]

    y = _in2f(x, in2f_W)

    idx = neighbors.astype(jnp.int32).reshape(-1)
    gathered = _sc_gather(y, idx)

    return _fused(dR_expanded, dR, pairwise_mask, gathered,
                  fW1, fb1.reshape(1, f), fW2, fb2.reshape(1, f),
                  f2out_W, f2out_b.reshape(1, d), out_W, out_b.reshape(1, d),
                  n, nbh, d, f, g)
